# Initial kernel scaffold; baseline (speedup 1.0000x reference)
import jax
import jax.numpy as jnp
from jax.experimental import pallas as pl


def kernel(h, triplets, W):
    raise NotImplementedError("write your pallas kernel here")


# speedup vs baseline: 3.3943x; 3.3943x over previous
import jax
import jax.numpy as jnp
from jax import lax
from jax.experimental import pallas as pl
from jax.experimental.pallas import tpu as pltpu
from jax.experimental.pallas import tpu_sc as plsc

_NC = 2
_NS = 16
_NW = _NC * _NS
_D = 128
_L = 16
_C = 80


def _sc_body(h_hbm, s_hbm, r_hbm, o_hbm, w_hbm, out_hbm,
             sidx, ridx, oidx, hs, wr, ho, outv, tmat, sem0, sem1, sem2):
    n = s_hbm.shape[0]
    t_per = n // _NW
    wid = lax.axis_index("s") * _NC + lax.axis_index("c")
    nchunks = t_per // _C

    @pl.loop(0, nchunks)
    def _chunk(g):
        base = wid * t_per + g * _C
        pltpu.sync_copy(s_hbm.at[pl.ds(base, _C)], sidx)
        pltpu.sync_copy(r_hbm.at[pl.ds(base, _C)], ridx)
        pltpu.sync_copy(o_hbm.at[pl.ds(base, _C)], oidx)
        c1 = pltpu.async_copy(h_hbm.at[sidx], hs, sem0)
        c2 = pltpu.async_copy(w_hbm.at[ridx], wr, sem1)
        c3 = pltpu.async_copy(h_hbm.at[oidx], ho, sem2)
        c1.wait()
        c2.wait()
        c3.wait()

        lane = lax.broadcasted_iota(jnp.int32, (_L,), 0)

        @pl.loop(0, _C // _L)
        def _grp(i):
            for j in range(_L):
                row = i * _L + j
                acc = jnp.zeros((_L,), jnp.float32)
                for k in range(_D // _L):
                    a = hs[row, pl.ds(k * _L, _L)]
                    b = wr[row, pl.ds(k * _L, _L)]
                    c = ho[row, pl.ds(k * _L, _L)]
                    acc = acc + a * b * c
                plsc.store_scatter(tmat, [lane, jnp.full((_L,), j, jnp.int32)],
                                   acc)
            res = jnp.zeros((_L,), jnp.float32)
            for l in range(_L):
                res = res + tmat[l, :]
            outv[pl.ds(i * _L, _L)] = res

        pltpu.sync_copy(outv, out_hbm.at[pl.ds(base, _C)])


def kernel(h, triplets, W):
    n = triplets.shape[0]
    assert n % (_NW * _C) == 0
    s = triplets[:, 0]
    r = triplets[:, 1]
    o = triplets[:, 2]
    mesh = plsc.VectorSubcoreMesh(
        core_axis_name="c", subcore_axis_name="s",
        num_cores=_NC, num_subcores=_NS)
    run = pl.kernel(
        _sc_body,
        out_type=jax.ShapeDtypeStruct((n,), jnp.float32),
        mesh=mesh,
        compiler_params=pltpu.CompilerParams(needs_layout_passes=False),
        scratch_types=[
            pltpu.VMEM((_C,), jnp.int32),
            pltpu.VMEM((_C,), jnp.int32),
            pltpu.VMEM((_C,), jnp.int32),
            pltpu.VMEM((_C, _D), jnp.float32),
            pltpu.VMEM((_C, _D), jnp.float32),
            pltpu.VMEM((_C, _D), jnp.float32),
            pltpu.VMEM((_C,), jnp.float32),
            pltpu.VMEM((_L, _L), jnp.float32),
            pltpu.SemaphoreType.DMA,
            pltpu.SemaphoreType.DMA,
            pltpu.SemaphoreType.DMA,
        ],
    )
    return run(h, s, r, o, W)


# speedup vs baseline: 7.5810x; 2.2334x over previous
import jax
import jax.numpy as jnp
from jax import lax
from jax.experimental import pallas as pl
from jax.experimental.pallas import tpu as pltpu
from jax.experimental.pallas import tpu_sc as plsc

_NC = 2
_NS = 16
_NW = _NC * _NS
_D = 128
_L = 16
_C = 80


def _sc_body(h_hbm, s_hbm, r_hbm, o_hbm, w_hbm, out_hbm,
             sidx, ridx, oidx, bufs, outs, tmat, gsems, osems):
    n = s_hbm.shape[0]
    t_per = n // _NW
    nch = t_per // _C
    wid = lax.axis_index("s") * _NC + lax.axis_index("c")
    gbase = wid * t_per

    pltpu.sync_copy(s_hbm.at[pl.ds(gbase, t_per)], sidx)
    pltpu.sync_copy(r_hbm.at[pl.ds(gbase, t_per)], ridx)
    pltpu.sync_copy(o_hbm.at[pl.ds(gbase, t_per)], oidx)

    def stage(g, b):
        base = g * _C
        hs, wr, ho = bufs[b]
        pltpu.async_copy(h_hbm.at[sidx.at[pl.ds(base, _C)]], hs, gsems[b])
        pltpu.async_copy(w_hbm.at[ridx.at[pl.ds(base, _C)]], wr, gsems[b])
        pltpu.async_copy(h_hbm.at[oidx.at[pl.ds(base, _C)]], ho, gsems[b])

    def drain_gather(b):
        hs, wr, ho = bufs[b]
        pltpu.make_async_copy(h_hbm.at[sidx.at[pl.ds(0, _C)]], hs,
                              gsems[b]).wait()
        pltpu.make_async_copy(w_hbm.at[ridx.at[pl.ds(0, _C)]], wr,
                              gsems[b]).wait()
        pltpu.make_async_copy(h_hbm.at[oidx.at[pl.ds(0, _C)]], ho,
                              gsems[b]).wait()

    def drain_out(b):
        pltpu.make_async_copy(outs[b], out_hbm.at[pl.ds(0, _C)],
                              osems[b]).wait()

    lane = lax.broadcasted_iota(jnp.int32, (_L,), 0)

    def compute(g, b, drain_pred):
        drain_gather(b)
        if isinstance(drain_pred, bool):
            if drain_pred:
                drain_out(b)
        else:
            @pl.when(drain_pred)
            def _():
                drain_out(b)
        hs, wr, ho = bufs[b]
        outv = outs[b]

        @pl.loop(0, _C // _L)
        def _grp(i):
            for j in range(_L):
                row = i * _L + j
                acc = jnp.zeros((_L,), jnp.float32)
                for k in range(_D // _L):
                    a = hs[row, pl.ds(k * _L, _L)]
                    b_ = wr[row, pl.ds(k * _L, _L)]
                    c = ho[row, pl.ds(k * _L, _L)]
                    acc = acc + a * b_ * c
                plsc.store_scatter(tmat, [lane, jnp.full((_L,), j, jnp.int32)],
                                   acc)
            res = jnp.zeros((_L,), jnp.float32)
            for l in range(_L):
                res = res + tmat[l, :]
            outv[pl.ds(i * _L, _L)] = res

        pltpu.async_copy(outv, out_hbm.at[pl.ds(gbase + g * _C, _C)], osems[b])

    stage(0, 0)

    @pl.loop(0, (nch - 1) // 2)
    def _pair(i):
        g = 2 * i
        stage(g + 1, 1)
        compute(g, 0, i > 0)
        stage(g + 2, 0)
        compute(g + 1, 1, i > 0)

    compute(nch - 1, 0, True)
    drain_out(0)
    drain_out(1)


def kernel(h, triplets, W):
    n = triplets.shape[0]
    assert n % (_NW * _C) == 0 and (n // (_NW * _C)) % 2 == 1
    s = triplets[:, 0]
    r = triplets[:, 1]
    o = triplets[:, 2]
    t_per = n // _NW
    mesh = plsc.VectorSubcoreMesh(
        core_axis_name="c", subcore_axis_name="s",
        num_cores=_NC, num_subcores=_NS)

    def body(h_hbm, s_hbm, r_hbm, o_hbm, w_hbm, out_hbm,
             sidx, ridx, oidx,
             hs0, wr0, ho0, hs1, wr1, ho1, out0, out1, tmat,
             gsem0, gsem1, osem0, osem1):
        _sc_body(h_hbm, s_hbm, r_hbm, o_hbm, w_hbm, out_hbm,
                 sidx, ridx, oidx,
                 [(hs0, wr0, ho0), (hs1, wr1, ho1)], [out0, out1], tmat,
                 [gsem0, gsem1], [osem0, osem1])

    run = pl.kernel(
        body,
        out_type=jax.ShapeDtypeStruct((n,), jnp.float32),
        mesh=mesh,
        compiler_params=pltpu.CompilerParams(needs_layout_passes=False),
        scratch_types=[
            pltpu.VMEM((t_per,), jnp.int32),
            pltpu.VMEM((t_per,), jnp.int32),
            pltpu.VMEM((t_per,), jnp.int32),
            pltpu.VMEM((_C, _D), jnp.float32),
            pltpu.VMEM((_C, _D), jnp.float32),
            pltpu.VMEM((_C, _D), jnp.float32),
            pltpu.VMEM((_C, _D), jnp.float32),
            pltpu.VMEM((_C, _D), jnp.float32),
            pltpu.VMEM((_C, _D), jnp.float32),
            pltpu.VMEM((_C,), jnp.float32),
            pltpu.VMEM((_C,), jnp.float32),
            pltpu.VMEM((_L, _L), jnp.float32),
            pltpu.SemaphoreType.DMA,
            pltpu.SemaphoreType.DMA,
            pltpu.SemaphoreType.DMA,
            pltpu.SemaphoreType.DMA,
        ],
    )
    return run(h, s, r, o, W)


# speedup vs baseline: 8.5211x; 1.1240x over previous
import jax
import jax.numpy as jnp
from jax import lax
from jax.experimental import pallas as pl
from jax.experimental.pallas import tpu as pltpu
from jax.experimental.pallas import tpu_sc as plsc

_NC = 2
_NS = 16
_NW = _NC * _NS
_D = 128
_L = 16
_C = 80


def _sc_body(h_hbm, s_hbm, r_hbm, o_hbm, w_hbm, out_hbm,
             sidx, ridx, oidx, bufs, outs, tmat, gsems, osems):
    n = s_hbm.shape[0]
    t_per = n // _NW
    nch = t_per // _C
    wid = lax.axis_index("s") * _NC + lax.axis_index("c")
    gbase = wid * t_per

    pltpu.sync_copy(s_hbm.at[pl.ds(gbase, t_per)], sidx)
    pltpu.sync_copy(r_hbm.at[pl.ds(gbase, t_per)], ridx)
    pltpu.sync_copy(o_hbm.at[pl.ds(gbase, t_per)], oidx)

    def stage(g, b):
        base = g * _C
        hs, wr, ho = bufs[b]
        pltpu.async_copy(h_hbm.at[sidx.at[pl.ds(base, _C)]], hs, gsems[b])
        pltpu.async_copy(w_hbm.at[ridx.at[pl.ds(base, _C)]], wr, gsems[b])
        pltpu.async_copy(h_hbm.at[oidx.at[pl.ds(base, _C)]], ho, gsems[b])

    def drain_gather(b):
        hs, wr, ho = bufs[b]
        pltpu.make_async_copy(h_hbm.at[sidx.at[pl.ds(0, _C)]], hs,
                              gsems[b]).wait()
        pltpu.make_async_copy(w_hbm.at[ridx.at[pl.ds(0, _C)]], wr,
                              gsems[b]).wait()
        pltpu.make_async_copy(h_hbm.at[oidx.at[pl.ds(0, _C)]], ho,
                              gsems[b]).wait()

    def drain_out(b):
        pltpu.make_async_copy(outs[b], out_hbm.at[pl.ds(0, _C)],
                              osems[b]).wait()

    lane = lax.broadcasted_iota(jnp.int32, (_L,), 0)

    def compute(g, b, drain_pred):
        drain_gather(b)
        if isinstance(drain_pred, bool):
            if drain_pred:
                drain_out(b)
        else:
            @pl.when(drain_pred)
            def _():
                drain_out(b)
        hs, wr, ho = bufs[b]
        outv = outs[b]

        @pl.loop(0, _C // _L)
        def _grp(i):
            for j in range(_L):
                row = i * _L + j
                acc = jnp.zeros((_L,), jnp.float32)
                for k in range(_D // (2 * _L)):
                    a = plsc.bitcast(hs[row, pl.ds(k * _L, _L)], jnp.bfloat16)
                    b_ = plsc.bitcast(wr[row, pl.ds(k * _L, _L)], jnp.bfloat16)
                    c = plsc.bitcast(ho[row, pl.ds(k * _L, _L)], jnp.bfloat16)
                    a0, a1 = plsc.unpack(a, format=plsc.PackFormat.INTERLEAVED)
                    b0, b1 = plsc.unpack(b_, format=plsc.PackFormat.INTERLEAVED)
                    c0, c1 = plsc.unpack(c, format=plsc.PackFormat.INTERLEAVED)
                    acc = acc + a0 * b0 * c0 + a1 * b1 * c1
                plsc.store_scatter(tmat, [lane, jnp.full((_L,), j, jnp.int32)],
                                   acc)
            res = jnp.zeros((_L,), jnp.float32)
            for l in range(_L):
                res = res + tmat[l, :]
            outv[pl.ds(i * _L, _L)] = res

        pltpu.async_copy(outv, out_hbm.at[pl.ds(gbase + g * _C, _C)], osems[b])

    stage(0, 0)

    @pl.loop(0, (nch - 1) // 2)
    def _pair(i):
        g = 2 * i
        stage(g + 1, 1)
        compute(g, 0, i > 0)
        stage(g + 2, 0)
        compute(g + 1, 1, i > 0)

    compute(nch - 1, 0, True)
    drain_out(0)
    drain_out(1)


def kernel(h, triplets, W):
    n = triplets.shape[0]
    assert n % (_NW * _C) == 0 and (n // (_NW * _C)) % 2 == 1
    s = triplets[:, 0]
    r = triplets[:, 1]
    o = triplets[:, 2]
    t_per = n // _NW
    mesh = plsc.VectorSubcoreMesh(
        core_axis_name="c", subcore_axis_name="s",
        num_cores=_NC, num_subcores=_NS)

    def body(h_hbm, s_hbm, r_hbm, o_hbm, w_hbm, out_hbm,
             sidx, ridx, oidx,
             hs0, wr0, ho0, hs1, wr1, ho1, out0, out1, tmat,
             gsem0, gsem1, osem0, osem1):
        _sc_body(h_hbm, s_hbm, r_hbm, o_hbm, w_hbm, out_hbm,
                 sidx, ridx, oidx,
                 [(hs0, wr0, ho0), (hs1, wr1, ho1)], [out0, out1], tmat,
                 [gsem0, gsem1], [osem0, osem1])

    run = pl.kernel(
        body,
        out_type=jax.ShapeDtypeStruct((n,), jnp.float32),
        mesh=mesh,
        compiler_params=pltpu.CompilerParams(needs_layout_passes=False,
                                             use_tc_tiling_on_sc=False),
        scratch_types=[
            pltpu.VMEM((t_per,), jnp.int32),
            pltpu.VMEM((t_per,), jnp.int32),
            pltpu.VMEM((t_per,), jnp.int32),
            pltpu.VMEM((_C, _D // 2), jnp.int32),
            pltpu.VMEM((_C, _D // 2), jnp.int32),
            pltpu.VMEM((_C, _D // 2), jnp.int32),
            pltpu.VMEM((_C, _D // 2), jnp.int32),
            pltpu.VMEM((_C, _D // 2), jnp.int32),
            pltpu.VMEM((_C, _D // 2), jnp.int32),
            pltpu.VMEM((_C,), jnp.float32),
            pltpu.VMEM((_C,), jnp.float32),
            pltpu.VMEM((_L, _L), jnp.float32),
            pltpu.SemaphoreType.DMA,
            pltpu.SemaphoreType.DMA,
            pltpu.SemaphoreType.DMA,
            pltpu.SemaphoreType.DMA,
        ],
    )
    def pack_bf16(x):
        xb = x.astype(jnp.bfloat16)
        return lax.bitcast_convert_type(
            xb.reshape(xb.shape[0], xb.shape[1] // 2, 2), jnp.int32)

    return run(pack_bf16(h), s, r, o, pack_bf16(W))


# speedup vs baseline: 8.5323x; 1.0013x over previous
import jax
import jax.numpy as jnp
from jax import lax
from jax.experimental import pallas as pl
from jax.experimental.pallas import tpu as pltpu
from jax.experimental.pallas import tpu_sc as plsc

_NC = 2
_NS = 16
_NW = _NC * _NS
_D = 128
_L = 16
_C = 80


def _sc_body(h_hbm, s_hbm, r_hbm, o_hbm, w_hbm, out_hbm,
             sidx, ridx, oidx, bufs, outs, tmat, gsems, osems):
    n = s_hbm.shape[0]
    t_per = n // _NW
    nch = t_per // _C
    wid = lax.axis_index("s") * _NC + lax.axis_index("c")
    gbase = wid * t_per

    pltpu.sync_copy(s_hbm.at[pl.ds(gbase, t_per)], sidx)
    pltpu.sync_copy(r_hbm.at[pl.ds(gbase, t_per)], ridx)
    pltpu.sync_copy(o_hbm.at[pl.ds(gbase, t_per)], oidx)

    def stage(g, b):
        base = g * _C
        hs, wr, ho = bufs[b]
        pltpu.async_copy(h_hbm.at[sidx.at[pl.ds(base, _C)]], hs, gsems[b])
        pltpu.async_copy(w_hbm.at[ridx.at[pl.ds(base, _C)]], wr, gsems[b])
        pltpu.async_copy(h_hbm.at[oidx.at[pl.ds(base, _C)]], ho, gsems[b])

    def drain_gather(b):
        hs, wr, ho = bufs[b]
        pltpu.make_async_copy(h_hbm.at[sidx.at[pl.ds(0, _C)]], hs,
                              gsems[b]).wait()
        pltpu.make_async_copy(w_hbm.at[ridx.at[pl.ds(0, _C)]], wr,
                              gsems[b]).wait()
        pltpu.make_async_copy(h_hbm.at[oidx.at[pl.ds(0, _C)]], ho,
                              gsems[b]).wait()

    def drain_out(b):
        pltpu.make_async_copy(outs[b], out_hbm.at[pl.ds(0, _C)],
                              osems[b]).wait()

    lane = lax.broadcasted_iota(jnp.int32, (_L,), 0)

    def compute(g, b, drain_pred):
        drain_gather(b)
        if isinstance(drain_pred, bool):
            if drain_pred:
                drain_out(b)
        else:
            @pl.when(drain_pred)
            def _():
                drain_out(b)
        hs, wr, ho = bufs[b]
        outv = outs[b]

        @pl.loop(0, _C // _L)
        def _grp(i):
            for j in range(_L):
                row = i * _L + j
                acc = jnp.zeros((_L,), jnp.float32)
                for k in range(_D // (2 * _L)):
                    a = plsc.bitcast(hs[row, pl.ds(k * _L, _L)], jnp.bfloat16)
                    b_ = plsc.bitcast(wr[row, pl.ds(k * _L, _L)], jnp.bfloat16)
                    c = plsc.bitcast(ho[row, pl.ds(k * _L, _L)], jnp.bfloat16)
                    p = a * b_ * c
                    p0, p1 = plsc.unpack(p, format=plsc.PackFormat.INTERLEAVED)
                    acc = acc + p0 + p1
                plsc.store_scatter(tmat, [lane, jnp.full((_L,), j, jnp.int32)],
                                   acc)
            res = jnp.zeros((_L,), jnp.float32)
            for l in range(_L):
                res = res + tmat[l, :]
            outv[pl.ds(i * _L, _L)] = res

        pltpu.async_copy(outv, out_hbm.at[pl.ds(gbase + g * _C, _C)], osems[b])

    stage(0, 0)

    @pl.loop(0, (nch - 1) // 2)
    def _pair(i):
        g = 2 * i
        stage(g + 1, 1)
        compute(g, 0, i > 0)
        stage(g + 2, 0)
        compute(g + 1, 1, i > 0)

    compute(nch - 1, 0, True)
    drain_out(0)
    drain_out(1)


def kernel(h, triplets, W):
    n = triplets.shape[0]
    assert n % (_NW * _C) == 0 and (n // (_NW * _C)) % 2 == 1
    s = triplets[:, 0]
    r = triplets[:, 1]
    o = triplets[:, 2]
    t_per = n // _NW
    mesh = plsc.VectorSubcoreMesh(
        core_axis_name="c", subcore_axis_name="s",
        num_cores=_NC, num_subcores=_NS)

    def body(h_hbm, s_hbm, r_hbm, o_hbm, w_hbm, out_hbm,
             sidx, ridx, oidx,
             hs0, wr0, ho0, hs1, wr1, ho1, out0, out1, tmat,
             gsem0, gsem1, osem0, osem1):
        _sc_body(h_hbm, s_hbm, r_hbm, o_hbm, w_hbm, out_hbm,
                 sidx, ridx, oidx,
                 [(hs0, wr0, ho0), (hs1, wr1, ho1)], [out0, out1], tmat,
                 [gsem0, gsem1], [osem0, osem1])

    run = pl.kernel(
        body,
        out_type=jax.ShapeDtypeStruct((n,), jnp.float32),
        mesh=mesh,
        compiler_params=pltpu.CompilerParams(needs_layout_passes=False,
                                             use_tc_tiling_on_sc=False),
        scratch_types=[
            pltpu.VMEM((t_per,), jnp.int32),
            pltpu.VMEM((t_per,), jnp.int32),
            pltpu.VMEM((t_per,), jnp.int32),
            pltpu.VMEM((_C, _D // 2), jnp.int32),
            pltpu.VMEM((_C, _D // 2), jnp.int32),
            pltpu.VMEM((_C, _D // 2), jnp.int32),
            pltpu.VMEM((_C, _D // 2), jnp.int32),
            pltpu.VMEM((_C, _D // 2), jnp.int32),
            pltpu.VMEM((_C, _D // 2), jnp.int32),
            pltpu.VMEM((_C,), jnp.float32),
            pltpu.VMEM((_C,), jnp.float32),
            pltpu.VMEM((_L, _L), jnp.float32),
            pltpu.SemaphoreType.DMA,
            pltpu.SemaphoreType.DMA,
            pltpu.SemaphoreType.DMA,
            pltpu.SemaphoreType.DMA,
        ],
    )
    def pack_bf16(x):
        xb = x.astype(jnp.bfloat16)
        return lax.bitcast_convert_type(
            xb.reshape(xb.shape[0], xb.shape[1] // 2, 2), jnp.int32)

    return run(pack_bf16(h), s, r, o, pack_bf16(W))
